# baseline (speedup 1.0000x reference)
import functools

import jax
import jax.numpy as jnp
from jax import lax
from jax.experimental import pallas as pl
from jax.experimental.pallas import tpu as pltpu
from jax.experimental.pallas import tpu_sc as plsc

N_NODES = 10000
N_PAD = 10240
DIM = 128
N_EDGES = 320000
N_SUBCORES = 16
CHUNK = 80
CNT_W = 128
ROWS_PER_TILE = N_PAD // N_SUBCORES
EDGES_PER_TILE = N_EDGES // N_SUBCORES
SUPER = 25
SUPERS_PER_TILE = EDGES_PER_TILE // CHUNK // SUPER
GCHUNK = 50
GSUPER = 20
GSUPERS = EDGES_PER_TILE // GCHUNK // GSUPER
NBUF = 4


def _sc_sums(job_h, machine_h, job_idx, machine_idx):
    tbl = jnp.stack([machine_h, job_h])
    gidx = jnp.stack([machine_idx, job_idx]).reshape(
        2, N_SUBCORES, GSUPERS, GSUPER, GCHUNK)
    sidx = jnp.stack([job_idx, machine_idx]).reshape(
        2, N_SUBCORES, GSUPERS, GSUPER, GCHUNK)
    zrows = jnp.zeros((ROWS_PER_TILE, DIM), jnp.float32)

    mesh = plsc.VectorSubcoreMesh(core_axis_name="c", subcore_axis_name="s")

    @functools.partial(
        pl.kernel,
        mesh=mesh,
        out_type=jax.ShapeDtypeStruct((2, N_PAD, DIM), jnp.float32),
        scratch_types=[
            pltpu.VMEM_SHARED((N_PAD, DIM), jnp.float32),
            pltpu.VMEM((GSUPER, GCHUNK), jnp.int32),
            pltpu.VMEM((GSUPER, GCHUNK), jnp.int32),
        ] + [pltpu.VMEM((GCHUNK, DIM), jnp.float32)] * NBUF
          + [pltpu.SemaphoreType.DMA] * (2 * NBUF),
    )
    def sums_kernel(tbl_hbm, gidx_hbm, sidx_hbm, zrows_hbm,
                    sum_out, acc, idx_g, idx_s, *bufsems):
        c = lax.axis_index("c")
        s = lax.axis_index("s")
        base = s * ROWS_PER_TILE
        rows_bufs = bufsems[:NBUF]
        gsems = bufsems[NBUF:2 * NBUF]
        ssems = bufsems[2 * NBUF:]
        bufs = tuple(zip(rows_bufs, gsems, ssems))

        pltpu.sync_copy(zrows_hbm, acc.at[pl.ds(base, ROWS_PER_TILE)])
        plsc.subcore_barrier()

        def super_body(u, _):
            pltpu.sync_copy(gidx_hbm.at[c, s, u], idx_g)
            pltpu.sync_copy(sidx_hbm.at[c, s, u], idx_s)
            for b, (rows, gsem, ssem) in enumerate(bufs):
                pltpu.async_copy(tbl_hbm.at[c].at[idx_g.at[b]], rows, gsem)

            def body(t, _):
                for b, (rows, gsem, ssem) in enumerate(bufs):
                    l = NBUF * t + b
                    pltpu.make_async_copy(
                        tbl_hbm.at[c].at[idx_g.at[l]], rows, gsem).wait()
                    pltpu.async_copy(rows, acc.at[idx_s.at[l]], ssem, add=True)

                    @pl.when(l + NBUF < GSUPER)
                    def _():
                        pltpu.make_async_copy(
                            rows, acc.at[idx_s.at[l]], ssem).wait()
                        pltpu.async_copy(
                            tbl_hbm.at[c].at[idx_g.at[l + NBUF]], rows, gsem)
                return 0
            lax.fori_loop(0, GSUPER // NBUF, body, 0)

            for b, (rows, gsem, ssem) in enumerate(bufs):
                pltpu.make_async_copy(
                    rows, acc.at[idx_s.at[GSUPER - NBUF + b]], ssem).wait()
            return 0
        lax.fori_loop(0, GSUPERS, super_body, 0)

        plsc.subcore_barrier()
        pltpu.sync_copy(acc.at[pl.ds(base, ROWS_PER_TILE)],
                        sum_out.at[c, pl.ds(base, ROWS_PER_TILE)])

    return sums_kernel(tbl, gidx, sidx, zrows)


def _sc_counts(job_idx, machine_idx):
    n_chunks = EDGES_PER_TILE // CHUNK
    sidx = jnp.stack([job_idx, machine_idx]).reshape(
        2, N_SUBCORES, n_chunks, CHUNK)
    zcnt = jnp.zeros((ROWS_PER_TILE, CNT_W), jnp.float32)
    ones = jnp.ones((CHUNK, CNT_W), jnp.float32)
    lag = 8

    mesh = plsc.VectorSubcoreMesh(core_axis_name="c", subcore_axis_name="s")

    @functools.partial(
        pl.kernel,
        mesh=mesh,
        out_type=jax.ShapeDtypeStruct((2, N_PAD, CNT_W), jnp.float32),
        scratch_types=[
            pltpu.VMEM_SHARED((N_PAD, CNT_W), jnp.float32),
            pltpu.VMEM((n_chunks, CHUNK), jnp.int32),
            pltpu.VMEM((CHUNK, CNT_W), jnp.float32),
            pltpu.SemaphoreType.DMA,
        ],
    )
    def cnt_kernel(sidx_hbm, ones_hbm, zcnt_hbm, cnt_out, cnt, idx_s, ones_v,
                   ssem):
        c = lax.axis_index("c")
        s = lax.axis_index("s")
        base = s * ROWS_PER_TILE

        pltpu.sync_copy(zcnt_hbm, cnt.at[pl.ds(base, ROWS_PER_TILE)])
        pltpu.sync_copy(ones_hbm, ones_v)
        pltpu.sync_copy(sidx_hbm.at[c, s], idx_s)
        plsc.subcore_barrier()

        def body(i, _):
            pltpu.async_copy(ones_v, cnt.at[idx_s.at[i]], ssem, add=True)

            @pl.when(i >= lag)
            def _():
                pltpu.make_async_copy(
                    ones_v, cnt.at[idx_s.at[i - lag]], ssem).wait()
            return 0
        lax.fori_loop(0, n_chunks, body, 0)

        def drain(i, _):
            pltpu.make_async_copy(
                ones_v, cnt.at[idx_s.at[n_chunks - lag + i]], ssem).wait()
            return 0
        lax.fori_loop(0, lag, drain, 0)

        plsc.subcore_barrier()
        pltpu.sync_copy(cnt.at[pl.ds(base, ROWS_PER_TILE)],
                        cnt_out.at[c, pl.ds(base, ROWS_PER_TILE)])

    return cnt_kernel(sidx, ones, zcnt)


def _matmul_body(h_ref, w_ref, b_ref, o_ref):
    o_ref[...] = jnp.dot(
        h_ref[...], w_ref[...], preferred_element_type=jnp.float32) + b_ref[...]


def _self_matmul(h, w1, b):
    n, d = h.shape
    out_dim = w1.shape[1]
    blk = 1000
    return pl.pallas_call(
        _matmul_body,
        grid=(n // blk,),
        in_specs=[
            pl.BlockSpec((blk, d), lambda i: (i, 0)),
            pl.BlockSpec((d, out_dim), lambda i: (0, 0)),
            pl.BlockSpec((1, out_dim), lambda i: (0, 0)),
        ],
        out_specs=pl.BlockSpec((blk, out_dim), lambda i: (i, 0)),
        out_shape=jax.ShapeDtypeStruct((n, out_dim), jnp.float32),
    )(h, w1, b.reshape(1, -1))


def _tail_body(x1_ref, sum_ref, cnt_ref, w2_ref, g_ref, be_ref, o_ref):
    deg = jnp.maximum(cnt_ref[0][:, 0:1], 1.0)
    agg = sum_ref[0] / deg
    x = x1_ref[...] + jnp.dot(agg, w2_ref[...],
                              preferred_element_type=jnp.float32)
    y = jnp.maximum(x, 0.0)
    m = jnp.mean(y, axis=1, keepdims=True)
    v = jnp.mean((y - m) ** 2, axis=1, keepdims=True)
    o_ref[...] = (y - m) * lax.rsqrt(v + 1e-5) * g_ref[...] + be_ref[...]


def _tail(x1, sums, cnts, side, w2, gamma, beta):
    n, out_dim = x1.shape
    d = sums.shape[2]
    blk = 1000
    return pl.pallas_call(
        _tail_body,
        grid=(n // blk,),
        in_specs=[
            pl.BlockSpec((blk, out_dim), lambda i: (i, 0)),
            pl.BlockSpec((1, blk, d), lambda i, side=side: (side, i, 0)),
            pl.BlockSpec((1, blk, CNT_W), lambda i, side=side: (side, i, 0)),
            pl.BlockSpec((d, out_dim), lambda i: (0, 0)),
            pl.BlockSpec((1, out_dim), lambda i: (0, 0)),
            pl.BlockSpec((1, out_dim), lambda i: (0, 0)),
        ],
        out_specs=pl.BlockSpec((blk, out_dim), lambda i: (i, 0)),
        out_shape=jax.ShapeDtypeStruct((n, out_dim), jnp.float32),
    )(x1, sums, cnts, w2, gamma.reshape(1, -1), beta.reshape(1, -1))


def kernel(job_h, machine_h, edge_index, W_job, b_job, W_machine, b_machine,
           gamma_j, beta_j, gamma_m, beta_m):
    job_idx = edge_index[0].astype(jnp.int32)
    machine_idx = edge_index[1].astype(jnp.int32)
    d = job_h.shape[1]

    x1_job = _self_matmul(job_h, W_job[:, :d].T, b_job)
    x1_machine = _self_matmul(machine_h, W_machine[:, :d].T, b_machine)

    sums = _sc_sums(job_h, machine_h, job_idx, machine_idx)
    cnts = _sc_counts(job_idx, machine_idx)

    new_job_h = _tail(x1_job, sums, cnts, 0, W_job[:, d:].T, gamma_j, beta_j)
    new_machine_h = _tail(x1_machine, sums, cnts, 1, W_machine[:, d:].T,
                          gamma_m, beta_m)
    return (new_job_h, new_machine_h)


# speedup vs baseline: 1.0869x; 1.0869x over previous
import functools

import jax
import jax.numpy as jnp
from jax import lax
from jax.experimental import pallas as pl
from jax.experimental.pallas import tpu as pltpu
from jax.experimental.pallas import tpu_sc as plsc

N_NODES = 10000
N_PAD = 10240
DIM = 128
N_EDGES = 320000
N_SUBCORES = 16
CHUNK = 80
CNT_W = 128
ROWS_PER_TILE = N_PAD // N_SUBCORES
EDGES_PER_TILE = N_EDGES // N_SUBCORES
SUPER = 25
SUPERS_PER_TILE = EDGES_PER_TILE // CHUNK // SUPER
GCHUNK = 80
GSUPER = 25
GSUPERS = EDGES_PER_TILE // GCHUNK // GSUPER
NBUF = 4


def _sc_sums(job_h, machine_h, job_idx, machine_idx):
    tbl = jnp.stack([machine_h, job_h])
    gidx = jnp.stack([machine_idx, job_idx]).reshape(
        2, N_SUBCORES, GSUPERS, GSUPER, GCHUNK)
    sidx = jnp.stack([job_idx, machine_idx]).reshape(
        2, N_SUBCORES, GSUPERS, GSUPER, GCHUNK)
    zrows = jnp.zeros((ROWS_PER_TILE, DIM), jnp.float32)

    mesh = plsc.VectorSubcoreMesh(core_axis_name="c", subcore_axis_name="s")

    @functools.partial(
        pl.kernel,
        mesh=mesh,
        out_type=jax.ShapeDtypeStruct((2, N_PAD, DIM), jnp.float32),
        scratch_types=[
            pltpu.VMEM_SHARED((N_PAD, DIM), jnp.float32),
            pltpu.VMEM((GSUPER, GCHUNK), jnp.int32),
            pltpu.VMEM((GSUPER, GCHUNK), jnp.int32),
        ] + [pltpu.VMEM((GCHUNK, DIM), jnp.float32)] * NBUF
          + [pltpu.SemaphoreType.DMA] * (2 * NBUF),
    )
    def sums_kernel(tbl_hbm, gidx_hbm, sidx_hbm, zrows_hbm,
                    sum_out, acc, idx_g, idx_s, *bufsems):
        c = lax.axis_index("c")
        s = lax.axis_index("s")
        base = s * ROWS_PER_TILE
        rows_bufs = bufsems[:NBUF]
        gsems = bufsems[NBUF:2 * NBUF]
        ssems = bufsems[2 * NBUF:]
        bufs = tuple(zip(rows_bufs, gsems, ssems))

        pltpu.sync_copy(zrows_hbm, acc.at[pl.ds(base, ROWS_PER_TILE)])
        plsc.subcore_barrier()

        def super_body(u, _):
            pltpu.sync_copy(gidx_hbm.at[c, s, u], idx_g)
            pltpu.sync_copy(sidx_hbm.at[c, s, u], idx_s)
            for b, (rows, gsem, ssem) in enumerate(bufs):
                pltpu.async_copy(tbl_hbm.at[c].at[idx_g.at[b]], rows, gsem)

            for l in range(GSUPER):
                rows, gsem, ssem = bufs[l % NBUF]
                pltpu.make_async_copy(
                    tbl_hbm.at[c].at[idx_g.at[l]], rows, gsem).wait()
                pltpu.async_copy(rows, acc.at[idx_s.at[l]], ssem, add=True)
                if l + NBUF < GSUPER:
                    pltpu.make_async_copy(
                        rows, acc.at[idx_s.at[l]], ssem).wait()
                    pltpu.async_copy(
                        tbl_hbm.at[c].at[idx_g.at[l + NBUF]], rows, gsem)

            for l in range(GSUPER - NBUF, GSUPER):
                rows, gsem, ssem = bufs[l % NBUF]
                pltpu.make_async_copy(
                    rows, acc.at[idx_s.at[l]], ssem).wait()
            return 0
        lax.fori_loop(0, GSUPERS, super_body, 0)

        plsc.subcore_barrier()
        pltpu.sync_copy(acc.at[pl.ds(base, ROWS_PER_TILE)],
                        sum_out.at[c, pl.ds(base, ROWS_PER_TILE)])

    return sums_kernel(tbl, gidx, sidx, zrows)


def _sc_counts(job_idx, machine_idx):
    n_chunks = EDGES_PER_TILE // CHUNK
    sidx = jnp.stack([job_idx, machine_idx]).reshape(
        2, N_SUBCORES, n_chunks, CHUNK)
    zcnt = jnp.zeros((ROWS_PER_TILE, CNT_W), jnp.float32)
    ones = jnp.ones((CHUNK, CNT_W), jnp.float32)
    lag = 8

    mesh = plsc.VectorSubcoreMesh(core_axis_name="c", subcore_axis_name="s")

    @functools.partial(
        pl.kernel,
        mesh=mesh,
        out_type=jax.ShapeDtypeStruct((2, N_PAD, CNT_W), jnp.float32),
        scratch_types=[
            pltpu.VMEM_SHARED((N_PAD, CNT_W), jnp.float32),
            pltpu.VMEM((n_chunks, CHUNK), jnp.int32),
            pltpu.VMEM((CHUNK, CNT_W), jnp.float32),
            pltpu.SemaphoreType.DMA,
        ],
    )
    def cnt_kernel(sidx_hbm, ones_hbm, zcnt_hbm, cnt_out, cnt, idx_s, ones_v,
                   ssem):
        c = lax.axis_index("c")
        s = lax.axis_index("s")
        base = s * ROWS_PER_TILE

        pltpu.sync_copy(zcnt_hbm, cnt.at[pl.ds(base, ROWS_PER_TILE)])
        pltpu.sync_copy(ones_hbm, ones_v)
        pltpu.sync_copy(sidx_hbm.at[c, s], idx_s)
        plsc.subcore_barrier()

        def body(i, _):
            pltpu.async_copy(ones_v, cnt.at[idx_s.at[i]], ssem, add=True)

            @pl.when(i >= lag)
            def _():
                pltpu.make_async_copy(
                    ones_v, cnt.at[idx_s.at[i - lag]], ssem).wait()
            return 0
        lax.fori_loop(0, n_chunks, body, 0)

        def drain(i, _):
            pltpu.make_async_copy(
                ones_v, cnt.at[idx_s.at[n_chunks - lag + i]], ssem).wait()
            return 0
        lax.fori_loop(0, lag, drain, 0)

        plsc.subcore_barrier()
        pltpu.sync_copy(cnt.at[pl.ds(base, ROWS_PER_TILE)],
                        cnt_out.at[c, pl.ds(base, ROWS_PER_TILE)])

    return cnt_kernel(sidx, ones, zcnt)


def _matmul_body(h_ref, w_ref, b_ref, o_ref):
    o_ref[...] = jnp.dot(
        h_ref[...], w_ref[...], preferred_element_type=jnp.float32) + b_ref[...]


def _self_matmul(h, w1, b):
    n, d = h.shape
    out_dim = w1.shape[1]
    blk = 1000
    return pl.pallas_call(
        _matmul_body,
        grid=(n // blk,),
        in_specs=[
            pl.BlockSpec((blk, d), lambda i: (i, 0)),
            pl.BlockSpec((d, out_dim), lambda i: (0, 0)),
            pl.BlockSpec((1, out_dim), lambda i: (0, 0)),
        ],
        out_specs=pl.BlockSpec((blk, out_dim), lambda i: (i, 0)),
        out_shape=jax.ShapeDtypeStruct((n, out_dim), jnp.float32),
    )(h, w1, b.reshape(1, -1))


def _tail_body(x1_ref, sum_ref, cnt_ref, w2_ref, g_ref, be_ref, o_ref):
    deg = jnp.maximum(cnt_ref[0][:, 0:1], 1.0)
    agg = sum_ref[0] / deg
    x = x1_ref[...] + jnp.dot(agg, w2_ref[...],
                              preferred_element_type=jnp.float32)
    y = jnp.maximum(x, 0.0)
    m = jnp.mean(y, axis=1, keepdims=True)
    v = jnp.mean((y - m) ** 2, axis=1, keepdims=True)
    o_ref[...] = (y - m) * lax.rsqrt(v + 1e-5) * g_ref[...] + be_ref[...]


def _tail(x1, sums, cnts, side, w2, gamma, beta):
    n, out_dim = x1.shape
    d = sums.shape[2]
    blk = 1000
    return pl.pallas_call(
        _tail_body,
        grid=(n // blk,),
        in_specs=[
            pl.BlockSpec((blk, out_dim), lambda i: (i, 0)),
            pl.BlockSpec((1, blk, d), lambda i, side=side: (side, i, 0)),
            pl.BlockSpec((1, blk, CNT_W), lambda i, side=side: (side, i, 0)),
            pl.BlockSpec((d, out_dim), lambda i: (0, 0)),
            pl.BlockSpec((1, out_dim), lambda i: (0, 0)),
            pl.BlockSpec((1, out_dim), lambda i: (0, 0)),
        ],
        out_specs=pl.BlockSpec((blk, out_dim), lambda i: (i, 0)),
        out_shape=jax.ShapeDtypeStruct((n, out_dim), jnp.float32),
    )(x1, sums, cnts, w2, gamma.reshape(1, -1), beta.reshape(1, -1))


def kernel(job_h, machine_h, edge_index, W_job, b_job, W_machine, b_machine,
           gamma_j, beta_j, gamma_m, beta_m):
    job_idx = edge_index[0].astype(jnp.int32)
    machine_idx = edge_index[1].astype(jnp.int32)
    d = job_h.shape[1]

    x1_job = _self_matmul(job_h, W_job[:, :d].T, b_job)
    x1_machine = _self_matmul(machine_h, W_machine[:, :d].T, b_machine)

    sums = _sc_sums(job_h, machine_h, job_idx, machine_idx)
    cnts = _sc_counts(job_idx, machine_idx)

    new_job_h = _tail(x1_job, sums, cnts, 0, W_job[:, d:].T, gamma_j, beta_j)
    new_machine_h = _tail(x1_machine, sums, cnts, 1, W_machine[:, d:].T,
                          gamma_m, beta_m)
    return (new_job_h, new_machine_h)
